# baseline (speedup 1.0000x reference)
import jax
import jax.numpy as jnp
from jax.experimental import pallas as pl
from jax.experimental.pallas import tpu as pltpu


def _round_up(n, m):
    return ((n + m - 1) // m) * m


def _linear_kernel(x_ref, w_ref, b_ref, o_ref):
    acc = jax.lax.dot_general(
        x_ref[...], w_ref[...], (((1,), (1,)), ((), ())),
        preferred_element_type=jnp.float32)
    o_ref[...] = acc + b_ref[...]


def _make_serial_stream_kernel(half):
    ch = half // 2

    def body(x_hbm, w_ref, b_ref, o_hbm, x_buf, o_buf, in_sem, out_sem):
        row0 = pl.program_id(0) * half

        def dma_in(step):
            return pltpu.make_async_copy(
                x_hbm.at[pl.ds(row0 + step * ch, ch), :],
                x_buf.at[step], in_sem.at[step])

        dma_in(0).start()
        wb = w_ref[...]
        brow = b_ref[...]
        dma_in(0).wait()
        dma_in(1).start()
        acc0 = jax.lax.dot_general(
            x_buf[0], wb, (((1,), (1,)), ((), ())),
            preferred_element_type=jnp.float32)
        o_buf[0:ch, :] = acc0 + brow
        dma_in(1).wait()
        acc1 = jax.lax.dot_general(
            x_buf[1], wb, (((1,), (1,)), ((), ())),
            preferred_element_type=jnp.float32)
        o_buf[ch:half, :] = acc1 + brow
        out_dma = pltpu.make_async_copy(
            o_buf, o_hbm.at[pl.ds(row0, half), :], out_sem)
        out_dma.start()
        out_dma.wait()

    return body


def kernel(x, weight, bias):
    B, D = x.shape
    C, D2 = weight.shape
    assert D == D2 and bias.shape == (C,)

    CPAD = _round_up(C, 128)

    TB = _round_up(B, 8)
    while TB > 8 and (2 * TB * (D + CPAD) * 4 + 2 * CPAD * D * 4
                      > 48 * 1024 * 1024 or TB * 2 > _round_up(B, 8)):
        TB = _round_up(TB // 2, 8)
    B_pad = _round_up(B, TB)

    x = x.astype(jnp.float32)
    x_p = x if B_pad == B else jnp.pad(x, ((0, B_pad - B), (0, 0)))
    w_p = weight.astype(jnp.float32)
    if CPAD != C:
        w_p = jnp.pad(w_p, ((0, CPAD - C), (0, 0)))
    b_row = jnp.pad(bias.astype(jnp.float32), (0, CPAD - C)).reshape(1, CPAD)

    cost = pl.CostEstimate(
        flops=2 * B * D * C,
        transcendentals=0,
        bytes_accessed=int(B_pad * D * 4 + D * CPAD * 4
                           + CPAD * 4 + B_pad * CPAD * 4),
    )

    half = B_pad // 2
    vmem_need = half * D * 4 + half * CPAD * 4 + CPAD * D * 4
    if B_pad % 4 == 0 and vmem_need <= 40 * 1024 * 1024:
        out_padded = pl.pallas_call(
            _make_serial_stream_kernel(half),
            out_shape=jax.ShapeDtypeStruct((B_pad, CPAD), jnp.float32),
            grid_spec=pltpu.PrefetchScalarGridSpec(
                num_scalar_prefetch=0,
                grid=(2,),
                in_specs=[
                    pl.BlockSpec(memory_space=pl.ANY),
                    pl.BlockSpec((CPAD, D), lambda i: (0, 0)),
                    pl.BlockSpec((1, CPAD), lambda i: (0, 0)),
                ],
                out_specs=pl.BlockSpec(memory_space=pl.ANY),
                scratch_shapes=[
                    pltpu.VMEM((2, half // 2, D), jnp.float32),
                    pltpu.VMEM((half, CPAD), jnp.float32),
                    pltpu.SemaphoreType.DMA((2,)),
                    pltpu.SemaphoreType.DMA,
                ],
            ),
            compiler_params=pltpu.CompilerParams(
                dimension_semantics=("parallel",),
                vmem_limit_bytes=56 * 1024 * 1024),
            cost_estimate=cost,
        )(x_p, w_p, b_row)
    else:
        out_padded = pl.pallas_call(
            _linear_kernel,
            out_shape=jax.ShapeDtypeStruct((B_pad, CPAD), jnp.float32),
            grid_spec=pltpu.PrefetchScalarGridSpec(
                num_scalar_prefetch=0,
                grid=(B_pad // TB,),
                in_specs=[
                    pl.BlockSpec((TB, D), lambda i: (i, 0)),
                    pl.BlockSpec((CPAD, D), lambda i: (0, 0)),
                    pl.BlockSpec((1, CPAD), lambda i: (0, 0)),
                ],
                out_specs=pl.BlockSpec((TB, CPAD), lambda i: (i, 0)),
            ),
            compiler_params=pltpu.CompilerParams(
                dimension_semantics=("parallel",),
                vmem_limit_bytes=56 * 1024 * 1024),
            cost_estimate=cost,
        )(x_p, w_p, b_row)

    return out_padded[:B, :C]


# speedup vs baseline: 1.1132x; 1.1132x over previous
import jax
import jax.numpy as jnp
from jax.experimental import pallas as pl
from jax.experimental.pallas import tpu as pltpu


def _round_up(n, m):
    return ((n + m - 1) // m) * m


def _linear_kernel(x_ref, w_ref, b_ref, o_ref):
    acc = jax.lax.dot_general(
        x_ref[...], w_ref[...], (((1,), (1,)), ((), ())),
        preferred_element_type=jnp.float32)
    o_ref[...] = acc + b_ref[...]


def _make_serial_stream_kernel(half):
    def body(x_hbm, w_ref, b_ref, o_hbm, x_buf, o_buf, in_sem, out_sem):
        row0 = pl.program_id(0) * half

        in_dma = pltpu.make_async_copy(
            x_hbm.at[pl.ds(row0, half), :], x_buf.at[0], in_sem.at[0])
        in_dma.start()
        wb = w_ref[...]
        brow = b_ref[...]
        in_dma.wait()
        acc = jax.lax.dot_general(
            x_buf[0], wb, (((1,), (1,)), ((), ())),
            preferred_element_type=jnp.float32)
        o_buf[...] = acc + brow
        out_dma = pltpu.make_async_copy(
            o_buf, o_hbm.at[pl.ds(row0, half), :], out_sem)
        out_dma.start()
        out_dma.wait()

    return body


def kernel(x, weight, bias):
    B, D = x.shape
    C, D2 = weight.shape
    assert D == D2 and bias.shape == (C,)

    CPAD = _round_up(C, 128)

    TB = _round_up(B, 8)
    while TB > 8 and (2 * TB * (D + CPAD) * 4 + 2 * CPAD * D * 4
                      > 48 * 1024 * 1024 or TB * 2 > _round_up(B, 8)):
        TB = _round_up(TB // 2, 8)
    B_pad = _round_up(B, TB)

    x = x.astype(jnp.float32)
    x_p = x if B_pad == B else jnp.pad(x, ((0, B_pad - B), (0, 0)))
    w_p = weight.astype(jnp.float32)
    if CPAD != C:
        w_p = jnp.pad(w_p, ((0, CPAD - C), (0, 0)))
    b_row = jnp.pad(bias.astype(jnp.float32), (0, CPAD - C)).reshape(1, CPAD)

    cost = pl.CostEstimate(
        flops=2 * B * D * C,
        transcendentals=0,
        bytes_accessed=int(B_pad * D * 4 + D * CPAD * 4
                           + CPAD * 4 + B_pad * CPAD * 4),
    )

    half = B_pad // 2
    vmem_need = half * D * 4 + half * CPAD * 4 + CPAD * D * 4
    if B_pad % 4 == 0 and vmem_need <= 40 * 1024 * 1024:
        out_padded = pl.pallas_call(
            _make_serial_stream_kernel(half),
            out_shape=jax.ShapeDtypeStruct((B_pad, CPAD), jnp.float32),
            grid_spec=pltpu.PrefetchScalarGridSpec(
                num_scalar_prefetch=0,
                grid=(2,),
                in_specs=[
                    pl.BlockSpec(memory_space=pl.ANY),
                    pl.BlockSpec((CPAD, D), lambda i: (0, 0)),
                    pl.BlockSpec((1, CPAD), lambda i: (0, 0)),
                ],
                out_specs=pl.BlockSpec(memory_space=pl.ANY),
                scratch_shapes=[
                    pltpu.VMEM((1, half, D), jnp.float32),
                    pltpu.VMEM((half, CPAD), jnp.float32),
                    pltpu.SemaphoreType.DMA((1,)),
                    pltpu.SemaphoreType.DMA,
                ],
            ),
            compiler_params=pltpu.CompilerParams(
                dimension_semantics=("parallel",),
                vmem_limit_bytes=56 * 1024 * 1024),
            cost_estimate=cost,
        )(x_p, w_p, b_row)
    else:
        out_padded = pl.pallas_call(
            _linear_kernel,
            out_shape=jax.ShapeDtypeStruct((B_pad, CPAD), jnp.float32),
            grid_spec=pltpu.PrefetchScalarGridSpec(
                num_scalar_prefetch=0,
                grid=(B_pad // TB,),
                in_specs=[
                    pl.BlockSpec((TB, D), lambda i: (i, 0)),
                    pl.BlockSpec((CPAD, D), lambda i: (0, 0)),
                    pl.BlockSpec((1, CPAD), lambda i: (0, 0)),
                ],
                out_specs=pl.BlockSpec((TB, CPAD), lambda i: (i, 0)),
            ),
            compiler_params=pltpu.CompilerParams(
                dimension_semantics=("parallel",),
                vmem_limit_bytes=56 * 1024 * 1024),
            cost_estimate=cost,
        )(x_p, w_p, b_row)

    return out_padded[:B, :C]


# speedup vs baseline: 1.3815x; 1.2410x over previous
import jax
import jax.numpy as jnp
from jax.experimental import pallas as pl
from jax.experimental.pallas import tpu as pltpu


def _round_up(n, m):
    return ((n + m - 1) // m) * m


def _make_kernel(TB, steps_per_core, CPAD):
    half = TB * steps_per_core

    def body(x_ref, w_ref, b_ref, o_hbm, o_buf, out_sem):
        i = pl.program_id(0)
        local = jax.lax.rem(i, steps_per_core)
        acc = jax.lax.dot_general(
            x_ref[...], w_ref[...], (((1,), (1,)), ((), ())),
            preferred_element_type=jnp.float32)
        o_buf[pl.ds(local * TB, TB), :] = acc + b_ref[...]

        @pl.when(local == steps_per_core - 1)
        def _():
            row0 = (i - local) * TB
            out_dma = pltpu.make_async_copy(
                o_buf, o_hbm.at[pl.ds(row0, half), :], out_sem)
            out_dma.start()
            out_dma.wait()

    return body


def kernel(x, weight, bias):
    B, D = x.shape
    C, D2 = weight.shape
    assert D == D2 and bias.shape == (C,)

    CPAD = _round_up(C, 128)

    TB = min(1024, _round_up(B, 8))
    if B >= 16 and _round_up(B, TB) // TB < 2:
        TB = _round_up((B + 1) // 2, 8)
    B_pad = _round_up(B, 2 * TB)
    n_blocks = B_pad // TB
    steps_per_core = n_blocks // 2

    x = x.astype(jnp.float32)
    x_p = x if B_pad == B else jnp.pad(x, ((0, B_pad - B), (0, 0)))
    w_p = weight.astype(jnp.float32)
    if CPAD != C:
        w_p = jnp.pad(w_p, ((0, CPAD - C), (0, 0)))
    b_row = jnp.pad(bias.astype(jnp.float32), (0, CPAD - C)).reshape(1, CPAD)

    cost = pl.CostEstimate(
        flops=2 * B * D * C,
        transcendentals=0,
        bytes_accessed=int(B_pad * D * 4 + D * CPAD * 4
                           + CPAD * 4 + B_pad * CPAD * 4),
    )

    out_padded = pl.pallas_call(
        _make_kernel(TB, steps_per_core, CPAD),
        out_shape=jax.ShapeDtypeStruct((B_pad, CPAD), jnp.float32),
        grid_spec=pltpu.PrefetchScalarGridSpec(
            num_scalar_prefetch=0,
            grid=(n_blocks,),
            in_specs=[
                pl.BlockSpec((TB, D), lambda i: (i, 0)),
                pl.BlockSpec((CPAD, D), lambda i: (0, 0)),
                pl.BlockSpec((1, CPAD), lambda i: (0, 0)),
            ],
            out_specs=pl.BlockSpec(memory_space=pl.ANY),
            scratch_shapes=[
                pltpu.VMEM((TB * steps_per_core, CPAD), jnp.float32),
                pltpu.SemaphoreType.DMA,
            ],
        ),
        compiler_params=pltpu.CompilerParams(
            dimension_semantics=("parallel",),
            vmem_limit_bytes=56 * 1024 * 1024),
        cost_estimate=cost,
    )(x_p, w_p, b_row)

    return out_padded[:B, :C]


# speedup vs baseline: 1.4791x; 1.0707x over previous
import jax
import jax.numpy as jnp
from jax.experimental import pallas as pl
from jax.experimental.pallas import tpu as pltpu


def _round_up(n, m):
    return ((n + m - 1) // m) * m


def _linear_kernel2(xa_ref, xb_ref, w_ref, b_ref, o_ref):
    TB = xa_ref.shape[0]
    acc_a = jax.lax.dot_general(
        xa_ref[...], w_ref[...], (((1,), (1,)), ((), ())),
        preferred_element_type=jnp.float32)
    o_ref[0:TB, :] = acc_a + b_ref[...]
    acc_b = jax.lax.dot_general(
        xb_ref[...], w_ref[...], (((1,), (1,)), ((), ())),
        preferred_element_type=jnp.float32)
    o_ref[TB:2 * TB, :] = acc_b + b_ref[...]


def kernel(x, weight, bias):
    B, D = x.shape
    C, D2 = weight.shape
    assert D == D2 and bias.shape == (C,)

    CPAD = _round_up(C, 128)
    B_pad = _round_up(B, 4)
    TB = B_pad // 4

    x = x.astype(jnp.float32)
    x_p = x if B_pad == B else jnp.pad(x, ((0, B_pad - B), (0, 0)))
    w_p = weight.astype(jnp.float32)
    if CPAD != C:
        w_p = jnp.pad(w_p, ((0, CPAD - C), (0, 0)))
    b_row = jnp.pad(bias.astype(jnp.float32), (0, CPAD - C)).reshape(1, CPAD)

    cost = pl.CostEstimate(
        flops=2 * B * D * C,
        transcendentals=0,
        bytes_accessed=int(B_pad * D * 4 + D * CPAD * 4
                           + CPAD * 4 + B_pad * CPAD * 4),
    )

    out_padded = pl.pallas_call(
        _linear_kernel2,
        out_shape=jax.ShapeDtypeStruct((B_pad, CPAD), jnp.float32),
        grid_spec=pltpu.PrefetchScalarGridSpec(
            num_scalar_prefetch=0,
            grid=(2,),
            in_specs=[
                pl.BlockSpec((TB, D), lambda i: (2 * i, 0)),
                pl.BlockSpec((TB, D), lambda i: (2 * i + 1, 0)),
                pl.BlockSpec((CPAD, D), lambda i: (0, 0)),
                pl.BlockSpec((1, CPAD), lambda i: (0, 0)),
            ],
            out_specs=pl.BlockSpec((2 * TB, CPAD), lambda i: (i, 0)),
        ),
        compiler_params=pltpu.CompilerParams(
            dimension_semantics=("parallel",),
            vmem_limit_bytes=56 * 1024 * 1024),
        cost_estimate=cost,
    )(x_p, x_p, w_p, b_row)

    return out_padded[:B, :C]


# speedup vs baseline: 1.4892x; 1.0068x over previous
import jax
import jax.numpy as jnp
from jax.experimental import pallas as pl
from jax.experimental.pallas import tpu as pltpu


def _round_up(n, m):
    return ((n + m - 1) // m) * m


def _linear_kernel(x_ref, w_ref, b_ref, o_ref):
    acc = jax.lax.dot_general(
        x_ref[...], w_ref[...], (((1,), (1,)), ((), ())),
        preferred_element_type=jnp.float32)
    o_ref[...] = acc + b_ref[...]


def kernel(x, weight, bias):
    B, D = x.shape
    C, D2 = weight.shape
    assert D == D2 and bias.shape == (C,)

    CPAD = _round_up(C, 128)

    TB = _round_up(B, 8)
    while TB > 8 and (2 * TB * (D + CPAD) * 4 + 2 * CPAD * D * 4
                      > 48 * 1024 * 1024 or TB * 2 > _round_up(B, 8)):
        TB = _round_up(TB // 2, 8)
    B_pad = _round_up(B, TB)

    x = x.astype(jnp.float32)
    x_p = x if B_pad == B else jnp.pad(x, ((0, B_pad - B), (0, 0)))
    w_p = weight.astype(jnp.float32)
    if CPAD != C:
        w_p = jnp.pad(w_p, ((0, CPAD - C), (0, 0)))
    b_row = jnp.pad(bias.astype(jnp.float32), (0, CPAD - C)).reshape(1, CPAD)

    cost = pl.CostEstimate(
        flops=2 * B * D * C,
        transcendentals=0,
        bytes_accessed=int(B_pad * D * 4 + D * CPAD * 4
                           + CPAD * 4 + B_pad * CPAD * 4),
    )

    out_padded = pl.pallas_call(
        _linear_kernel,
        out_shape=jax.ShapeDtypeStruct((B_pad, CPAD), jnp.float32),
        grid_spec=pltpu.PrefetchScalarGridSpec(
            num_scalar_prefetch=0,
            grid=(B_pad // TB,),
            in_specs=[
                pl.BlockSpec((TB, D), lambda i: (i, 0)),
                pl.BlockSpec((CPAD, D), lambda i: (0, 0)),
                pl.BlockSpec((1, CPAD), lambda i: (0, 0)),
            ],
            out_specs=pl.BlockSpec((TB, CPAD), lambda i: (i, 0)),
        ),
        compiler_params=pltpu.CompilerParams(
            dimension_semantics=("parallel",),
            vmem_limit_bytes=56 * 1024 * 1024),
        cost_estimate=cost,
    )(x_p, w_p, b_row)

    return out_padded[:B, :C]
